# baseline (speedup 1.0000x reference)
import functools

import jax
import jax.numpy as jnp
from jax import lax
from jax.experimental import pallas as pl
from jax.experimental.pallas import tpu as pltpu
from jax.experimental.pallas import tpu_sc as plsc

_B4 = 16384
_F = 64
_NW = 32
_NPW = _B4 // _NW
_CHUNK = 128
_NCH = _NPW // _CHUNK
_NG = _NPW // 16

_mesh = plsc.VectorSubcoreMesh(core_axis_name="c", subcore_axis_name="s")


@functools.partial(
    pl.kernel,
    mesh=_mesh,
    compiler_params=pltpu.CompilerParams(
        needs_layout_passes=False, use_tc_tiling_on_sc=False),
    out_type=jax.ShapeDtypeStruct((_B4,), jnp.float32),
    scratch_types=[
        pltpu.VMEM((_NCH, _CHUNK), jnp.int32),
        pltpu.VMEM((_NCH, _CHUNK), jnp.int32),
        pltpu.VMEM((_NPW,), jnp.int32),
        pltpu.VMEM((_NPW, _F), jnp.float32),
        pltpu.VMEM((_NPW, _F), jnp.float32),
        pltpu.VMEM((2 * _F,), jnp.float32),
        pltpu.VMEM((_NPW * 16,), jnp.float32),
        pltpu.VMEM((_NPW,), jnp.float32),
        pltpu.SemaphoreType.DMA,
    ],
)
def _sc_score(head_hbm, tail_hbm, rel_hbm, table_hbm, r_hbm, out_hbm,
              hidx, tidx, ridx, hrows, trows, r_v, pbuf, outv, sem):
    wid = lax.axis_index("s") * 2 + lax.axis_index("c")
    base = wid * _NPW

    pltpu.sync_copy(head_hbm.at[pl.ds(wid * _NCH, _NCH)], hidx)
    pltpu.sync_copy(tail_hbm.at[pl.ds(wid * _NCH, _NCH)], tidx)
    pltpu.sync_copy(rel_hbm.at[pl.ds(base, _NPW)], ridx)
    pltpu.sync_copy(r_hbm, r_v)

    copies = []
    for j in range(_NCH):
        copies.append(pltpu.async_copy(
            table_hbm.at[hidx.at[j]], hrows.at[pl.ds(j * _CHUNK, _CHUNK)],
            sem))
        copies.append(pltpu.async_copy(
            table_hbm.at[tidx.at[j]], trows.at[pl.ds(j * _CHUNK, _CHUNK)],
            sem))
    for cp in copies:
        cp.wait()

    r0 = [r_v[pl.ds(c * 16, 16)] for c in range(4)]
    r1 = [r_v[pl.ds(_F + c * 16, 16)] for c in range(4)]

    def passa(g, carry):
        relv = ridx[pl.ds(g * 16, 16)]
        for k in range(16):
            i = g * 16 + k
            is1 = relv[k] != 0
            acc4 = jnp.zeros((16,), jnp.float32)
            for c in range(4):
                hv = hrows[i, pl.ds(c * 16, 16)]
                tv = trows[i, pl.ds(c * 16, 16)]
                rv = jnp.where(is1, r1[c], r0[c])
                s = hv - tv + rv
                acc4 = acc4 + s * s
            pbuf[pl.ds(i * 16, 16)] = acc4
        return carry

    lax.fori_loop(0, _NG, passa, 0)

    iota16 = lax.iota(jnp.int32, 16)

    def group(g, carry):
        lanebase = g * 256 + iota16 * 16
        acc = jnp.zeros((16,), jnp.float32)
        for j in range(16):
            acc = acc + plsc.load_gather(pbuf, [lanebase + j])
        outv[pl.ds(g * 16, 16)] = -acc
        return carry

    lax.fori_loop(0, _NG, group, 0)

    pltpu.sync_copy(outv, out_hbm.at[pl.ds(base, _NPW)])


def kernel(item_embedding, r_weight, head_ids, tail_ids, relation_ids):
    heads = head_ids.reshape(_NW * _NCH, _CHUNK).astype(jnp.int32)
    tails = tail_ids.reshape(_NW * _NCH, _CHUNK).astype(jnp.int32)
    rels = relation_ids.reshape(_B4).astype(jnp.int32)
    r_flat = r_weight.reshape(2 * _F)
    out = _sc_score(heads, tails, rels, item_embedding, r_flat)
    return out.reshape(head_ids.shape)
